# baseline (speedup 1.0000x reference)
import functools

import jax
import jax.numpy as jnp
from jax import lax
from jax.experimental import pallas as pl
from jax.experimental.pallas import tpu as pltpu
from jax.experimental.pallas import tpu_sc as plsc

N = 2
LQ = 5440
LQP = 5632
C = 256
NH = 8
NL = 4
NP = 4
DH = 32
QB = 512
NBLK = LQP // QB
NCH = LQP // 128
LEN = 5440
SHAPES = ((64, 64), (32, 32), (16, 16), (8, 8))
STARTS = (0, 4096, 5120, 5376)
CORNERS = ((0, 0), (1, 0), (0, 1), (1, 1))


def _lane_consts():
    j = lax.broadcasted_iota(jnp.int32, (1, 128), 1)
    lvec = (j // NP) % NL
    wl = jnp.zeros((1, 128), jnp.float32)
    hl = jnp.zeros((1, 128), jnp.float32)
    sv = jnp.zeros((1, 128), jnp.float32)
    for l in range(NL):
        wl = jnp.where(lvec == l, float(SHAPES[l][1]), wl)
        hl = jnp.where(lvec == l, float(SHAPES[l][0]), hl)
        sv = jnp.where(lvec == l, float(STARTS[l]), sv)
    return lvec, wl, hl, sv


def _t(x):
    return jnp.transpose(x)


def _dot(a, b):
    return jnp.dot(a, b, preferred_element_type=jnp.float32,
                   precision=lax.Precision.HIGHEST)


def _phase1_body(q_ref, v_ref, rx_ref, ry_ref, wv_ref, bv_ref, wox_ref,
                 box_ref, woy_ref, boy_ref, wa_ref, ba_ref,
                 val_ref, idx_ref, w_ref):
    q = q_ref[0]
    v = v_ref[0]
    val = _dot(v, wv_ref[...]) + bv_ref[...]

    def rnd(x):
        bb = lax.bitcast_convert_type(x, jnp.int32)
        lsb = lax.shift_right_logical(bb, 16) & 1
        return bb + 32767 + lsb

    lob = lax.shift_right_logical(rnd(val[:, :128]), 16)
    hib = rnd(val[:, 128:]) & jnp.int32(-65536)
    word = hib | lob
    val_ref[0] = _t(lax.bitcast_convert_type(word, jnp.float32))
    offx = _dot(q, wox_ref[...]) + box_ref[...]
    offy = _dot(q, woy_ref[...]) + boy_ref[...]
    la = _dot(q, wa_ref[...]) + ba_ref[...]
    m = jnp.max(la, axis=-1, keepdims=True)
    e = jnp.exp(la - m)
    bi = lax.broadcasted_iota(jnp.int32, (128, 128), 0) // 16
    bj = lax.broadcasted_iota(jnp.int32, (128, 128), 1) // 16
    bd = (bi == bj).astype(jnp.float32)
    denom = _dot(e, bd)
    attnw = e / denom
    lvec, wl, hl, sv = _lane_consts()
    rxq = rx_ref[0]
    ryq = ry_ref[0]
    rxb = jnp.zeros((QB, 128), jnp.float32)
    ryb = jnp.zeros((QB, 128), jnp.float32)
    for l in range(NL):
        rxb = jnp.where(lvec == l, rxq[:, l:l + 1], rxb)
        ryb = jnp.where(lvec == l, ryq[:, l:l + 1], ryb)
    x = rxb * wl + offx - 0.5
    y = ryb * hl + offy - 0.5
    x0 = jnp.floor(x)
    fx = x - x0
    y0 = jnp.floor(y)
    fy = y - y0
    for c, (dx, dy) in enumerate(CORNERS):
        cx = x0 + dx
        cy = y0 + dy
        valid = ((cx >= 0) & (cx <= wl - 1) & (cy >= 0) & (cy <= hl - 1)).astype(jnp.float32)
        xc = jnp.clip(cx, 0.0, wl - 1)
        yc = jnp.clip(cy, 0.0, hl - 1)
        pos = sv + yc * wl + xc
        wx = fx if dx else (1.0 - fx)
        wy = fy if dy else (1.0 - fy)
        idx_ref[0, c] = jnp.clip(_t(pos).astype(jnp.int32), 0, LEN - 1)
        w_ref[0, c] = _t(wx * wy * valid * attnw)


def _phase1(qpad, vpad, rx, ry, Wv, bv, Woffx, boffx, Woffy, boffy, Wattn, battn):
    full = lambda s: pl.BlockSpec(s, lambda n, b: (0,) * len(s))
    blk = lambda s: pl.BlockSpec(s, lambda n, b: (n, b) + (0,) * (len(s) - 2))
    tblk = lambda s: pl.BlockSpec(s, lambda n, b: (n,) + (0,) * (len(s) - 2) + (b,))
    return pl.pallas_call(
        _phase1_body,
        grid=(N, NBLK),
        in_specs=[blk((1, QB, C)), blk((1, QB, C)), blk((1, QB, NL)),
                  blk((1, QB, NL)), full((C, C)), full((1, C)),
                  full((C, 128)), full((1, 128)), full((C, 128)),
                  full((1, 128)), full((C, 128)), full((1, 128))],
        out_specs=[tblk((1, C // 2, QB)), tblk((1, 4, 128, QB)), tblk((1, 4, 128, QB))],
        out_shape=[jax.ShapeDtypeStruct((N, C // 2, LQP), jnp.float32),
                   jax.ShapeDtypeStruct((N, 4, 128, LQP), jnp.int32),
                   jax.ShapeDtypeStruct((N, 4, 128, LQP), jnp.float32)],
        compiler_params=pltpu.CompilerParams(
            fuse_transposed_lhs_in_matmul=True),
    )(qpad, vpad, rx, ry, Wv, bv, Woffx, boffx, Woffy, boffy, Wattn, battn)


def _sc_body(value_hbm, idx_hbm, w_hbm, out_hbm,
             table_v, idx_v, w_v, out_v, isem, wsem, osem):
    n = lax.axis_index("c")
    s = lax.axis_index("s")
    h = s // 2
    half = s % 2
    r0 = h * DH + half * 16
    pltpu.sync_copy(value_hbm.at[n, pl.ds(h * 16 + half * 8, 8), :], table_v)

    def in_copies(ch, b):
        q0 = ch * 128
        return (pltpu.make_async_copy(
                    idx_hbm.at[n, :, pl.ds(h * 16, 16), pl.ds(q0, 128)],
                    idx_v.at[b], isem.at[b]),
                pltpu.make_async_copy(
                    w_hbm.at[n, :, pl.ds(h * 16, 16), pl.ds(q0, 128)],
                    w_v.at[b], wsem.at[b]))

    def out_copy(ch, b):
        q0 = ch * 128
        return pltpu.make_async_copy(
            out_v.at[b], out_hbm.at[n, pl.ds(r0, 16), pl.ds(q0, 128)], osem.at[b])

    for c in in_copies(0, 0):
        c.start()

    def chunk(ch, b):
        nxt = in_copies(ch + 1, 1 - b)

        @pl.when(ch + 1 < NCH)
        def _():
            for c in nxt:
                c.start()

        for c in in_copies(ch, b):
            c.wait()

        @pl.when(ch >= 2)
        def _():
            out_copy(ch - 2, b).wait()

        for sub in range(8):
            jvec = sub * 16 + lax.iota(jnp.int32, 16)

            def sbody(grp, accs):
                pacc = [jnp.zeros((32,), jnp.bfloat16)] * 8
                for i in range(8):
                    si = grp * 8 + i
                    cvec = jnp.full((16,), si // 16, jnp.int32)
                    lpvec = jnp.full((16,), si % 16, jnp.int32)
                    pos = plsc.load_gather(idx_v.at[b], [cvec, lpvec, jvec])
                    wgt = plsc.load_gather(w_v.at[b], [cvec, lpvec, jvec])
                    wgtp = plsc.pack(wgt, wgt,
                                     format=plsc.PackFormat.INTERLEAVED)
                    for k in range(8):
                        g = plsc.load_gather(
                            table_v, [jnp.full((16,), k, jnp.int32), pos])
                        pacc[k] = pacc[k] + plsc.bitcast(g, jnp.bfloat16) * wgtp
                new = []
                for k in range(8):
                    ae, ao = plsc.unpack(pacc[k],
                                         format=plsc.PackFormat.INTERLEAVED)
                    new.append(accs[2 * k] + ae)
                    new.append(accs[2 * k + 1] + ao)
                return tuple(new)

            zero = jnp.zeros((16,), jnp.float32)
            accs = lax.fori_loop(0, 8, sbody, (zero,) * 16)
            for d in range(16):
                out_v[b, d, pl.ds(sub * 16, 16)] = accs[d]
        out_copy(ch, b).start()

    def chunk2(ch0, _):
        for b in range(2):
            chunk(ch0 * 2 + b, b)
        return 0

    lax.fori_loop(0, NCH // 2, chunk2, 0)
    for b in range(2):
        out_copy(NCH - 2 + b, b).wait()


@functools.cache
def _sc_gather():
    return functools.partial(
        pl.kernel,
        mesh=plsc.VectorSubcoreMesh(core_axis_name="c", subcore_axis_name="s"),
        out_type=jax.ShapeDtypeStruct((N, C, LQP), jnp.float32),
        compiler_params=pltpu.CompilerParams(needs_layout_passes=False),
        scratch_types=[pltpu.VMEM((8, LQP), jnp.float32),
                       pltpu.VMEM((2, 4, 16, 128), jnp.int32),
                       pltpu.VMEM((2, 4, 16, 128), jnp.float32),
                       pltpu.VMEM((2, 16, 128), jnp.float32),
                       pltpu.SemaphoreType.DMA((2,)),
                       pltpu.SemaphoreType.DMA((2,)),
                       pltpu.SemaphoreType.DMA((2,))],
    )(_sc_body)


def _phase3_body(r_ref, wo_ref, bo_ref, o_ref):
    o_ref[0] = lax.dot_general(
        r_ref[0], wo_ref[...], (((0,), (0,)), ((), ())),
        preferred_element_type=jnp.float32,
        precision=lax.Precision.HIGHEST) + bo_ref[...]


def _phase3(rows_t, Wout, bout):
    full = lambda s: pl.BlockSpec(s, lambda n, b: (0,) * len(s))
    return pl.pallas_call(
        _phase3_body,
        grid=(N, NBLK),
        in_specs=[pl.BlockSpec((1, C, QB), lambda n, b: (n, 0, b)),
                  full((C, C)), full((1, C))],
        out_specs=pl.BlockSpec((1, QB, C), lambda n, b: (n, b, 0)),
        out_shape=jax.ShapeDtypeStruct((N, LQP, C), jnp.float32),
        compiler_params=pltpu.CompilerParams(
            fuse_transposed_lhs_in_matmul=True),
    )(rows_t, Wout, bout.reshape(1, C))


def kernel(query, reference_points, input_flatten, input_spatial_shapes,
           input_level_start_index, Wv, bv, Woff, boff, Wattn, battn,
           Wout, bout):
    Woffx = Woff.reshape(C, NH * NL * NP, 2)[:, :, 0]
    Woffy = Woff.reshape(C, NH * NL * NP, 2)[:, :, 1]
    boffx = boff.reshape(1, NH * NL * NP, 2)[:, :, 0]
    boffy = boff.reshape(1, NH * NL * NP, 2)[:, :, 1]
    pad = ((0, 0), (0, LQP - LQ), (0, 0))
    qpad = jnp.pad(query, pad)
    vpad = jnp.pad(input_flatten, pad)
    rx = jnp.pad(reference_points[..., 0], pad)
    ry = jnp.pad(reference_points[..., 1], pad)

    Wv = jnp.concatenate([Wv[:, 0::2], Wv[:, 1::2]], axis=1)
    bv = jnp.concatenate([bv[0::2], bv[1::2]])
    value_t, idx_t, w_t = _phase1(qpad, vpad, rx, ry, Wv, bv.reshape(1, C),
                                  Woffx, boffx, Woffy, boffy, Wattn,
                                  battn.reshape(1, NH * NL * NP))

    rows_t = _sc_gather()(value_t, idx_t, w_t)

    return _phase3(rows_t, Wout, bout)[:, :LQ, :]


# speedup vs baseline: 1.1447x; 1.1447x over previous
import functools

import jax
import jax.numpy as jnp
from jax import lax
from jax.experimental import pallas as pl
from jax.experimental.pallas import tpu as pltpu
from jax.experimental.pallas import tpu_sc as plsc

N = 2
LQ = 5440
LQP = 5632
C = 256
NH = 8
NL = 4
NP = 4
DH = 32
QB = 512
NBLK = LQP // QB
NCH = LQP // 128
LEN = 5440
SHAPES = ((64, 64), (32, 32), (16, 16), (8, 8))
STARTS = (0, 4096, 5120, 5376)
CORNERS = ((0, 0), (1, 0), (0, 1), (1, 1))


def _lane_consts():
    j = lax.broadcasted_iota(jnp.int32, (1, 128), 1)
    lvec = (j // NP) % NL
    wl = jnp.zeros((1, 128), jnp.float32)
    hl = jnp.zeros((1, 128), jnp.float32)
    sv = jnp.zeros((1, 128), jnp.float32)
    for l in range(NL):
        wl = jnp.where(lvec == l, float(SHAPES[l][1]), wl)
        hl = jnp.where(lvec == l, float(SHAPES[l][0]), hl)
        sv = jnp.where(lvec == l, float(STARTS[l]), sv)
    return lvec, wl, hl, sv


def _t(x):
    return jnp.transpose(x)


def _dot(a, b):
    return jnp.dot(a, b, preferred_element_type=jnp.float32,
                   precision=lax.Precision.HIGHEST)


def _phase1_body(q_ref, v_ref, rx_ref, ry_ref, wv_ref, bv_ref, wox_ref,
                 box_ref, woy_ref, boy_ref, wa_ref, ba_ref,
                 val_ref, idx_ref, w_ref):
    q = q_ref[0]
    v = v_ref[0]
    val = _dot(v, wv_ref[...]) + bv_ref[...]

    def rnd(x):
        bb = lax.bitcast_convert_type(x, jnp.int32)
        lsb = lax.shift_right_logical(bb, 16) & 1
        return bb + 32767 + lsb

    lob = lax.shift_right_logical(rnd(val[:, :128]), 16)
    hib = rnd(val[:, 128:]) & jnp.int32(-65536)
    word = hib | lob
    val_ref[0] = _t(lax.bitcast_convert_type(word, jnp.float32))
    offx = _dot(q, wox_ref[...]) + box_ref[...]
    offy = _dot(q, woy_ref[...]) + boy_ref[...]
    la = _dot(q, wa_ref[...]) + ba_ref[...]
    m = jnp.max(la, axis=-1, keepdims=True)
    e = jnp.exp(la - m)
    bi = lax.broadcasted_iota(jnp.int32, (128, 128), 0) // 16
    bj = lax.broadcasted_iota(jnp.int32, (128, 128), 1) // 16
    bd = (bi == bj).astype(jnp.float32)
    denom = _dot(e, bd)
    attnw = e / denom
    lvec, wl, hl, sv = _lane_consts()
    rxq = rx_ref[0]
    ryq = ry_ref[0]
    rxb = jnp.zeros((QB, 128), jnp.float32)
    ryb = jnp.zeros((QB, 128), jnp.float32)
    for l in range(NL):
        rxb = jnp.where(lvec == l, rxq[:, l:l + 1], rxb)
        ryb = jnp.where(lvec == l, ryq[:, l:l + 1], ryb)
    x = rxb * wl + offx - 0.5
    y = ryb * hl + offy - 0.5
    x0 = jnp.floor(x)
    fx = x - x0
    y0 = jnp.floor(y)
    fy = y - y0
    for c, (dx, dy) in enumerate(CORNERS):
        cx = x0 + dx
        cy = y0 + dy
        valid = ((cx >= 0) & (cx <= wl - 1) & (cy >= 0) & (cy <= hl - 1)).astype(jnp.float32)
        xc = jnp.clip(cx, 0.0, wl - 1)
        yc = jnp.clip(cy, 0.0, hl - 1)
        pos = sv + yc * wl + xc
        wx = fx if dx else (1.0 - fx)
        wy = fy if dy else (1.0 - fy)
        idx_ref[0, c] = jnp.clip(_t(pos).astype(jnp.int32), 0, LEN - 1)
        w_ref[0, c] = _t(wx * wy * valid * attnw)


def _phase1(qpad, vpad, rx, ry, Wv, bv, Woffx, boffx, Woffy, boffy, Wattn, battn):
    full = lambda s: pl.BlockSpec(s, lambda n, b: (0,) * len(s))
    blk = lambda s: pl.BlockSpec(s, lambda n, b: (n, b) + (0,) * (len(s) - 2))
    tblk = lambda s: pl.BlockSpec(s, lambda n, b: (n,) + (0,) * (len(s) - 2) + (b,))
    return pl.pallas_call(
        _phase1_body,
        grid=(N, NBLK),
        in_specs=[blk((1, QB, C)), blk((1, QB, C)), blk((1, QB, NL)),
                  blk((1, QB, NL)), full((C, C)), full((1, C)),
                  full((C, 128)), full((1, 128)), full((C, 128)),
                  full((1, 128)), full((C, 128)), full((1, 128))],
        out_specs=[tblk((1, C // 2, QB)), tblk((1, 4, 128, QB)), tblk((1, 4, 128, QB))],
        out_shape=[jax.ShapeDtypeStruct((N, C // 2, LQP), jnp.float32),
                   jax.ShapeDtypeStruct((N, 4, 128, LQP), jnp.int32),
                   jax.ShapeDtypeStruct((N, 4, 128, LQP), jnp.float32)],
        compiler_params=pltpu.CompilerParams(
            fuse_transposed_lhs_in_matmul=True),
    )(qpad, vpad, rx, ry, Wv, bv, Woffx, boffx, Woffy, boffy, Wattn, battn)


def _sc_body(value_hbm, idx_hbm, w_hbm, out_hbm,
             table_v, idx_v, w_v, out_v, isem, wsem, osem):
    n = lax.axis_index("c")
    s = lax.axis_index("s")
    h = s // 2
    half = s % 2
    r0 = h * DH + half * 16
    pltpu.sync_copy(value_hbm.at[n, pl.ds(h * 16 + half * 8, 8), :], table_v)

    def in_copies(ch, b):
        q0 = ch * 128
        return (pltpu.make_async_copy(
                    idx_hbm.at[n, :, pl.ds(h * 16, 16), pl.ds(q0, 128)],
                    idx_v.at[b], isem.at[b]),
                pltpu.make_async_copy(
                    w_hbm.at[n, :, pl.ds(h * 16, 16), pl.ds(q0, 128)],
                    w_v.at[b], wsem.at[b]))

    def out_copy(ch, b):
        q0 = ch * 128
        return pltpu.make_async_copy(
            out_v.at[b], out_hbm.at[n, pl.ds(r0, 16), pl.ds(q0, 128)], osem.at[b])

    for c in in_copies(0, 0):
        c.start()

    def chunk(ch, b):
        nxt = in_copies(ch + 1, 1 - b)

        @pl.when(ch + 1 < NCH)
        def _():
            for c in nxt:
                c.start()

        for c in in_copies(ch, b):
            c.wait()

        @pl.when(ch >= 2)
        def _():
            out_copy(ch - 2, b).wait()

        for sub in range(8):
            jvec = sub * 16 + lax.iota(jnp.int32, 16)

            def sbody(grp, accs):
                pacc = [jnp.zeros((32,), jnp.bfloat16)] * 8
                for i in range(4):
                    si = grp * 4 + i
                    cvec = jnp.full((16,), si // 16, jnp.int32)
                    lpvec = jnp.full((16,), si % 16, jnp.int32)
                    pos = plsc.load_gather(idx_v.at[b], [cvec, lpvec, jvec])
                    wgt = plsc.load_gather(w_v.at[b], [cvec, lpvec, jvec])
                    wgtp = plsc.pack(wgt, wgt,
                                     format=plsc.PackFormat.INTERLEAVED)
                    for k in range(8):
                        g = plsc.load_gather(
                            table_v, [jnp.full((16,), k, jnp.int32), pos])
                        pacc[k] = pacc[k] + plsc.bitcast(g, jnp.bfloat16) * wgtp
                new = []
                for k in range(8):
                    ae, ao = plsc.unpack(pacc[k],
                                         format=plsc.PackFormat.INTERLEAVED)
                    new.append(accs[2 * k] + ae)
                    new.append(accs[2 * k + 1] + ao)
                return tuple(new)

            zero = jnp.zeros((16,), jnp.float32)
            accs = lax.fori_loop(0, 16, sbody, (zero,) * 16)
            for d in range(16):
                out_v[b, d, pl.ds(sub * 16, 16)] = accs[d]
        out_copy(ch, b).start()

    def chunk2(ch0, _):
        for b in range(2):
            chunk(ch0 * 2 + b, b)
        return 0

    lax.fori_loop(0, NCH // 2, chunk2, 0)
    for b in range(2):
        out_copy(NCH - 2 + b, b).wait()


@functools.cache
def _sc_gather():
    return functools.partial(
        pl.kernel,
        mesh=plsc.VectorSubcoreMesh(core_axis_name="c", subcore_axis_name="s"),
        out_type=jax.ShapeDtypeStruct((N, C, LQP), jnp.float32),
        compiler_params=pltpu.CompilerParams(needs_layout_passes=False),
        scratch_types=[pltpu.VMEM((8, LQP), jnp.float32),
                       pltpu.VMEM((2, 4, 16, 128), jnp.int32),
                       pltpu.VMEM((2, 4, 16, 128), jnp.float32),
                       pltpu.VMEM((2, 16, 128), jnp.float32),
                       pltpu.SemaphoreType.DMA((2,)),
                       pltpu.SemaphoreType.DMA((2,)),
                       pltpu.SemaphoreType.DMA((2,))],
    )(_sc_body)


def _phase3_body(r_ref, wo_ref, bo_ref, o_ref):
    o_ref[0] = lax.dot_general(
        r_ref[0], wo_ref[...], (((0,), (0,)), ((), ())),
        preferred_element_type=jnp.float32,
        precision=lax.Precision.HIGHEST) + bo_ref[...]


def _phase3(rows_t, Wout, bout):
    full = lambda s: pl.BlockSpec(s, lambda n, b: (0,) * len(s))
    return pl.pallas_call(
        _phase3_body,
        grid=(N, NBLK),
        in_specs=[pl.BlockSpec((1, C, QB), lambda n, b: (n, 0, b)),
                  full((C, C)), full((1, C))],
        out_specs=pl.BlockSpec((1, QB, C), lambda n, b: (n, b, 0)),
        out_shape=jax.ShapeDtypeStruct((N, LQ, C), jnp.float32),
        compiler_params=pltpu.CompilerParams(
            fuse_transposed_lhs_in_matmul=True),
    )(rows_t, Wout, bout.reshape(1, C))


def kernel(query, reference_points, input_flatten, input_spatial_shapes,
           input_level_start_index, Wv, bv, Woff, boff, Wattn, battn,
           Wout, bout):
    Woffx = Woff.reshape(C, NH * NL * NP, 2)[:, :, 0]
    Woffy = Woff.reshape(C, NH * NL * NP, 2)[:, :, 1]
    boffx = boff.reshape(1, NH * NL * NP, 2)[:, :, 0]
    boffy = boff.reshape(1, NH * NL * NP, 2)[:, :, 1]
    qpad = query
    vpad = input_flatten
    rx = reference_points[..., 0]
    ry = reference_points[..., 1]

    Wv = jnp.concatenate([Wv[:, 0::2], Wv[:, 1::2]], axis=1)
    bv = jnp.concatenate([bv[0::2], bv[1::2]])
    value_t, idx_t, w_t = _phase1(qpad, vpad, rx, ry, Wv, bv.reshape(1, C),
                                  Woffx, boffx, Woffy, boffy, Wattn,
                                  battn.reshape(1, NH * NL * NP))

    rows_t = _sc_gather()(value_t, idx_t, w_t)

    return _phase3(rows_t, Wout, bout)


# speedup vs baseline: 1.1750x; 1.0265x over previous
import functools

import jax
import jax.numpy as jnp
from jax import lax
from jax.experimental import pallas as pl
from jax.experimental.pallas import tpu as pltpu
from jax.experimental.pallas import tpu_sc as plsc

N = 2
LQ = 5440
LQP = 5632
C = 256
NH = 8
NL = 4
NP = 4
DH = 32
QB = 1408
NBLK = LQP // QB
NCH = LQP // 128
LEN = 5440
SHAPES = ((64, 64), (32, 32), (16, 16), (8, 8))
STARTS = (0, 4096, 5120, 5376)
CORNERS = ((0, 0), (1, 0), (0, 1), (1, 1))


def _lane_consts():
    j = lax.broadcasted_iota(jnp.int32, (1, 128), 1)
    lvec = (j // NP) % NL
    wl = jnp.zeros((1, 128), jnp.float32)
    hl = jnp.zeros((1, 128), jnp.float32)
    sv = jnp.zeros((1, 128), jnp.float32)
    for l in range(NL):
        wl = jnp.where(lvec == l, float(SHAPES[l][1]), wl)
        hl = jnp.where(lvec == l, float(SHAPES[l][0]), hl)
        sv = jnp.where(lvec == l, float(STARTS[l]), sv)
    return lvec, wl, hl, sv


def _t(x):
    return jnp.transpose(x)


def _dot(a, b):
    return jnp.dot(a, b, preferred_element_type=jnp.float32,
                   precision=lax.Precision.HIGHEST)


def _phase1_body(q_ref, v_ref, rx_ref, ry_ref, wv_ref, bv_ref, wox_ref,
                 box_ref, woy_ref, boy_ref, wa_ref, ba_ref,
                 val_ref, idx_ref, w_ref):
    q = q_ref[0]
    v = v_ref[0]
    val = _dot(v, wv_ref[...]) + bv_ref[...]

    def rnd(x):
        bb = lax.bitcast_convert_type(x, jnp.int32)
        lsb = lax.shift_right_logical(bb, 16) & 1
        return bb + 32767 + lsb

    lob = lax.shift_right_logical(rnd(val[:, :128]), 16)
    hib = rnd(val[:, 128:]) & jnp.int32(-65536)
    word = hib | lob
    val_ref[0] = _t(lax.bitcast_convert_type(word, jnp.float32))
    offx = _dot(q, wox_ref[...]) + box_ref[...]
    offy = _dot(q, woy_ref[...]) + boy_ref[...]
    la = _dot(q, wa_ref[...]) + ba_ref[...]
    m = jnp.max(la, axis=-1, keepdims=True)
    e = jnp.exp(la - m)
    bi = lax.broadcasted_iota(jnp.int32, (128, 128), 0) // 16
    bj = lax.broadcasted_iota(jnp.int32, (128, 128), 1) // 16
    bd = (bi == bj).astype(jnp.float32)
    denom = _dot(e, bd)
    attnw = e / denom
    lvec, wl, hl, sv = _lane_consts()
    rxq = rx_ref[0]
    ryq = ry_ref[0]
    rxb = jnp.zeros((QB, 128), jnp.float32)
    ryb = jnp.zeros((QB, 128), jnp.float32)
    for l in range(NL):
        rxb = jnp.where(lvec == l, rxq[:, l:l + 1], rxb)
        ryb = jnp.where(lvec == l, ryq[:, l:l + 1], ryb)
    x = rxb * wl + offx - 0.5
    y = ryb * hl + offy - 0.5
    x0 = jnp.floor(x)
    fx = x - x0
    y0 = jnp.floor(y)
    fy = y - y0
    for c, (dx, dy) in enumerate(CORNERS):
        cx = x0 + dx
        cy = y0 + dy
        valid = ((cx >= 0) & (cx <= wl - 1) & (cy >= 0) & (cy <= hl - 1)).astype(jnp.float32)
        xc = jnp.clip(cx, 0.0, wl - 1)
        yc = jnp.clip(cy, 0.0, hl - 1)
        pos = sv + yc * wl + xc
        wx = fx if dx else (1.0 - fx)
        wy = fy if dy else (1.0 - fy)
        idx_ref[0, c] = jnp.clip(_t(pos).astype(jnp.int32), 0, LEN - 1)
        w_ref[0, c] = _t(wx * wy * valid * attnw)


def _phase1(qpad, vpad, rx, ry, Wv, bv, Woffx, boffx, Woffy, boffy, Wattn, battn):
    full = lambda s: pl.BlockSpec(s, lambda n, b: (0,) * len(s))
    blk = lambda s: pl.BlockSpec(s, lambda n, b: (n, b) + (0,) * (len(s) - 2))
    tblk = lambda s: pl.BlockSpec(s, lambda n, b: (n,) + (0,) * (len(s) - 2) + (b,))
    return pl.pallas_call(
        _phase1_body,
        grid=(N, NBLK),
        in_specs=[blk((1, QB, C)), blk((1, QB, C)), blk((1, QB, NL)),
                  blk((1, QB, NL)), full((C, C)), full((1, C)),
                  full((C, 128)), full((1, 128)), full((C, 128)),
                  full((1, 128)), full((C, 128)), full((1, 128))],
        out_specs=[tblk((1, C // 2, QB)), tblk((1, 4, 128, QB)), tblk((1, 4, 128, QB))],
        out_shape=[jax.ShapeDtypeStruct((N, C // 2, LQP), jnp.float32),
                   jax.ShapeDtypeStruct((N, 4, 128, LQP), jnp.int32),
                   jax.ShapeDtypeStruct((N, 4, 128, LQP), jnp.float32)],
        compiler_params=pltpu.CompilerParams(
            fuse_transposed_lhs_in_matmul=True),
    )(qpad, vpad, rx, ry, Wv, bv, Woffx, boffx, Woffy, boffy, Wattn, battn)


def _sc_body(value_hbm, idx_hbm, w_hbm, out_hbm,
             table_v, idx_v, w_v, out_v, isem, wsem, osem):
    n = lax.axis_index("c")
    s = lax.axis_index("s")
    h = s // 2
    half = s % 2
    r0 = h * DH + half * 16
    pltpu.sync_copy(value_hbm.at[n, pl.ds(h * 16 + half * 8, 8), :], table_v)

    def in_copies(ch, b):
        q0 = ch * 128
        return (pltpu.make_async_copy(
                    idx_hbm.at[n, :, pl.ds(h * 16, 16), pl.ds(q0, 128)],
                    idx_v.at[b], isem.at[b]),
                pltpu.make_async_copy(
                    w_hbm.at[n, :, pl.ds(h * 16, 16), pl.ds(q0, 128)],
                    w_v.at[b], wsem.at[b]))

    def out_copy(ch, b):
        q0 = ch * 128
        return pltpu.make_async_copy(
            out_v.at[b], out_hbm.at[n, pl.ds(r0, 16), pl.ds(q0, 128)], osem.at[b])

    for c in in_copies(0, 0):
        c.start()

    def chunk(ch, b):
        nxt = in_copies(ch + 1, 1 - b)

        @pl.when(ch + 1 < NCH)
        def _():
            for c in nxt:
                c.start()

        for c in in_copies(ch, b):
            c.wait()

        @pl.when(ch >= 2)
        def _():
            out_copy(ch - 2, b).wait()

        for sub in range(8):
            jvec = sub * 16 + lax.iota(jnp.int32, 16)

            def sbody(grp, accs):
                pacc = [jnp.zeros((32,), jnp.bfloat16)] * 8
                for i in range(4):
                    si = grp * 4 + i
                    cvec = jnp.full((16,), si // 16, jnp.int32)
                    lpvec = jnp.full((16,), si % 16, jnp.int32)
                    pos = plsc.load_gather(idx_v.at[b], [cvec, lpvec, jvec])
                    wgt = plsc.load_gather(w_v.at[b], [cvec, lpvec, jvec])
                    wgtp = plsc.pack(wgt, wgt,
                                     format=plsc.PackFormat.INTERLEAVED)
                    for k in range(8):
                        g = plsc.load_gather(
                            table_v, [jnp.full((16,), k, jnp.int32), pos])
                        pacc[k] = pacc[k] + plsc.bitcast(g, jnp.bfloat16) * wgtp
                new = []
                for k in range(8):
                    ae, ao = plsc.unpack(pacc[k],
                                         format=plsc.PackFormat.INTERLEAVED)
                    new.append(accs[2 * k] + ae)
                    new.append(accs[2 * k + 1] + ao)
                return tuple(new)

            zero = jnp.zeros((16,), jnp.float32)
            accs = lax.fori_loop(0, 16, sbody, (zero,) * 16)
            for d in range(16):
                out_v[b, d, pl.ds(sub * 16, 16)] = accs[d]
        out_copy(ch, b).start()

    def chunk2(ch0, _):
        for b in range(2):
            chunk(ch0 * 2 + b, b)
        return 0

    lax.fori_loop(0, NCH // 2, chunk2, 0)
    for b in range(2):
        out_copy(NCH - 2 + b, b).wait()


@functools.cache
def _sc_gather():
    return functools.partial(
        pl.kernel,
        mesh=plsc.VectorSubcoreMesh(core_axis_name="c", subcore_axis_name="s"),
        out_type=jax.ShapeDtypeStruct((N, C, LQP), jnp.float32),
        compiler_params=pltpu.CompilerParams(needs_layout_passes=False),
        scratch_types=[pltpu.VMEM((8, LQP), jnp.float32),
                       pltpu.VMEM((2, 4, 16, 128), jnp.int32),
                       pltpu.VMEM((2, 4, 16, 128), jnp.float32),
                       pltpu.VMEM((2, 16, 128), jnp.float32),
                       pltpu.SemaphoreType.DMA((2,)),
                       pltpu.SemaphoreType.DMA((2,)),
                       pltpu.SemaphoreType.DMA((2,))],
    )(_sc_body)


def _phase3_body(r_ref, wo_ref, bo_ref, o_ref):
    o_ref[0] = lax.dot_general(
        r_ref[0], wo_ref[...], (((0,), (0,)), ((), ())),
        preferred_element_type=jnp.float32,
        precision=lax.Precision.HIGHEST) + bo_ref[...]


def _phase3(rows_t, Wout, bout):
    full = lambda s: pl.BlockSpec(s, lambda n, b: (0,) * len(s))
    return pl.pallas_call(
        _phase3_body,
        grid=(N, NBLK),
        in_specs=[pl.BlockSpec((1, C, QB), lambda n, b: (n, 0, b)),
                  full((C, C)), full((1, C))],
        out_specs=pl.BlockSpec((1, QB, C), lambda n, b: (n, b, 0)),
        out_shape=jax.ShapeDtypeStruct((N, LQ, C), jnp.float32),
        compiler_params=pltpu.CompilerParams(
            fuse_transposed_lhs_in_matmul=True),
    )(rows_t, Wout, bout.reshape(1, C))


def kernel(query, reference_points, input_flatten, input_spatial_shapes,
           input_level_start_index, Wv, bv, Woff, boff, Wattn, battn,
           Wout, bout):
    Woffx = Woff.reshape(C, NH * NL * NP, 2)[:, :, 0]
    Woffy = Woff.reshape(C, NH * NL * NP, 2)[:, :, 1]
    boffx = boff.reshape(1, NH * NL * NP, 2)[:, :, 0]
    boffy = boff.reshape(1, NH * NL * NP, 2)[:, :, 1]
    qpad = query
    vpad = input_flatten
    rx = reference_points[..., 0]
    ry = reference_points[..., 1]

    Wv = jnp.concatenate([Wv[:, 0::2], Wv[:, 1::2]], axis=1)
    bv = jnp.concatenate([bv[0::2], bv[1::2]])
    value_t, idx_t, w_t = _phase1(qpad, vpad, rx, ry, Wv, bv.reshape(1, C),
                                  Woffx, boffx, Woffy, boffy, Wattn,
                                  battn.reshape(1, NH * NL * NP))

    rows_t = _sc_gather()(value_t, idx_t, w_t)

    return _phase3(rows_t, Wout, bout)
